# Initial kernel scaffold; baseline (speedup 1.0000x reference)
import jax
import jax.numpy as jnp
from jax.experimental import pallas as pl


def kernel(neighbors, static, lanes, lanes_speed_limit, lanes_has_speed_limit, routes, W_agent, b_agent, type_emb, W_static, b_static, W_lane, b_lane, W_sl, b_sl, unknown_sl, traffic_emb, W_route, b_route, W_pos, b_pos):
    raise NotImplementedError("write your pallas kernel here")


# speedup vs baseline: 1.6853x; 1.6853x over previous
import jax
import jax.numpy as jnp
from jax.experimental import pallas as pl

_B = 512
_N = 32
_T = 21
_AD = 11
_S = 5
_SD = 10
_L = 70
_P = 20
_LD = 12
_R = 25
_H = 256
_ACT = 8
_PRED = 10
_TOK = _N + _S + _L + _ACT + _PRED
_BB = 8


def _body(nb_ref, st_ref, ln_ref, sl_ref, hs_ref, rt_ref,
          xc_ref, yc_ref, xr_ref, yr_ref,
          Wa_ref, ba_ref, temb_ref, Ws_ref, bs_ref, Wl_ref, bl_ref,
          Wsl_ref, bsl_ref, usl_ref, tremb_ref, Wr_ref, br_ref,
          Wp_ref, bp_ref,
          enc_a_ref, ln_enc_ref, mask_a_ref, ln_valid_ref,
          rcond_ref, tdist_ref):
    f32 = jnp.float32
    Wp = Wp_ref[...]
    bp = bp_ref[...]

    nbf = nb_ref[...]
    valid_nb = (jnp.sum(jnp.abs(nbf), axis=1, keepdims=True) > 0.0).astype(f32)
    tidx = (jnp.abs(nbf[:, 230:231]) * 997.0).astype(jnp.int32) % 5
    oh_t = (tidx == jax.lax.broadcasted_iota(jnp.int32, (_BB * _N, 5), 1)).astype(f32)
    enc_nb = (jnp.dot(nbf, Wa_ref[...], preferred_element_type=f32)
              + ba_ref[...]
              + jnp.dot(oh_t, temb_ref[...], preferred_element_type=f32))
    pe_nb = jnp.dot(nbf[:, 220:227], Wp, preferred_element_type=f32) + bp
    enc_nb = enc_nb + valid_nb * pe_nb

    stf = st_ref[...]
    valid_st = (jnp.sum(jnp.abs(stf), axis=1, keepdims=True) > 0.0).astype(f32)
    enc_st = jnp.dot(stf, Ws_ref[...], preferred_element_type=f32) + bs_ref[...]
    pe_st = jnp.dot(stf[:, 0:7], Wp, preferred_element_type=f32) + bp
    enc_st = enc_st + valid_st * pe_st

    lnf = ln_ref[...]
    valid_ln = (jnp.sum(jnp.abs(lnf), axis=1, keepdims=True) > 0.0).astype(f32)
    tr_idx = (jnp.abs(lnf[:, 11:12]) * 997.0).astype(jnp.int32) % 4
    oh_tr = (tr_idx == jax.lax.broadcasted_iota(jnp.int32, (_BB * _L, 4), 1)).astype(f32)
    sl_emb = sl_ref[...] * Wsl_ref[...] + bsl_ref[...]
    sl_emb = jnp.where(hs_ref[...] > 0.5, sl_emb, usl_ref[...])
    enc_ln = (jnp.dot(lnf, Wl_ref[...], preferred_element_type=f32)
              + bl_ref[...] + sl_emb
              + jnp.dot(oh_tr, tremb_ref[...], preferred_element_type=f32))
    pe_ln = jnp.dot(lnf[:, 120:127], Wp, preferred_element_type=f32) + bp
    enc_ln = enc_ln + valid_ln * pe_ln

    h = jnp.tanh(jnp.dot(rt_ref[...], Wr_ref[...], preferred_element_type=f32)
                 + br_ref[...])
    row = jax.lax.broadcasted_iota(jnp.int32, (_BB, _BB * _R), 1)
    grp = jax.lax.broadcasted_iota(jnp.int32, (_BB, _BB * _R), 0)
    sel = (row // _R == grp).astype(f32) * (1.0 / _R)
    rcond_ref[...] = jnp.dot(sel, h, preferred_element_type=f32)

    dx = xc_ref[...] - xr_ref[...]
    dy = yc_ref[...] - yr_ref[...]
    tdist_ref[...] = jnp.sqrt(dx * dx + dy * dy)

    for k in range(_BB):
        enc_a_ref[k, 0:_N, :] = enc_nb[k * _N:(k + 1) * _N, :]
        enc_a_ref[k, _N:_N + _S, :] = enc_st[k * _S:(k + 1) * _S, :]
        mask_a_ref[k, 0:_N, :] = valid_nb[k * _N:(k + 1) * _N, :]
        mask_a_ref[k, _N:_N + _S, :] = valid_st[k * _S:(k + 1) * _S, :]
        ln_enc_ref[k, :, :] = enc_ln[k * _L:(k + 1) * _L, :]
        ln_valid_ref[k, :, :] = valid_ln[k * _L:(k + 1) * _L, :]


def kernel(neighbors, static, lanes, lanes_speed_limit, lanes_has_speed_limit,
           routes, W_agent, b_agent, type_emb, W_static, b_static, W_lane,
           b_lane, W_sl, b_sl, unknown_sl, traffic_emb, W_route, b_route,
           W_pos, b_pos):
    f32 = jnp.float32
    Bc = neighbors.shape[0]
    nb2 = neighbors.reshape(Bc * _N, _T * _AD)
    st2 = static.reshape(Bc * _S, _SD)
    ln2 = lanes.reshape(Bc * _L, _P * _LD)
    sl2 = lanes_speed_limit.reshape(Bc * _L, 1)
    hs2 = lanes_has_speed_limit.reshape(Bc * _L, 1).astype(f32)
    rt2 = routes.reshape(Bc * _R, _P * _LD)

    nloc = neighbors[:, :, -1, :2]
    sloc = static[:, :, :2]
    lloc = lanes[:, :, _P // 2, :2]
    eloc = jnp.tile(jnp.array([-0.5, 0.0], f32)[None, None, :], (Bc, _ACT, 1))
    ploc = neighbors[:, :_PRED, -1, :2]
    all_loc = jnp.concatenate([nloc, sloc, lloc, eloc, ploc], axis=1)
    xc = all_loc[:, :, 0:1]
    yc = all_loc[:, :, 1:2]
    xr = all_loc[:, :, 0].reshape(Bc, 1, _TOK)
    yr = all_loc[:, :, 1].reshape(Bc, 1, _TOK)

    grid = Bc // _BB
    K = _T * _AD
    KL = _P * _LD

    def bm(*shape):
        nd = len(shape)
        return pl.BlockSpec(shape, lambda i, nd=nd: (i,) + (0,) * (nd - 1))

    def full(*shape):
        nd = len(shape)
        return pl.BlockSpec(shape, lambda i, nd=nd: (0,) * nd)

    out = pl.pallas_call(
        _body,
        grid=(grid,),
        in_specs=[
            bm(_BB * _N, K), bm(_BB * _S, _SD), bm(_BB * _L, KL),
            bm(_BB * _L, 1), bm(_BB * _L, 1), bm(_BB * _R, KL),
            bm(_BB, _TOK, 1), bm(_BB, _TOK, 1),
            bm(_BB, 1, _TOK), bm(_BB, 1, _TOK),
            full(K, _H), full(1, _H), full(5, _H), full(_SD, _H),
            full(1, _H), full(KL, _H), full(1, _H), full(1, _H),
            full(1, _H), full(1, _H), full(4, _H), full(KL, _H),
            full(1, _H), full(7, _H), full(1, _H),
        ],
        out_specs=[
            bm(_BB, _N + _S, _H), bm(_BB, _L, _H),
            bm(_BB, _N + _S, 1), bm(_BB, _L, 1),
            bm(_BB, _H), bm(_BB, _TOK, _TOK),
        ],
        out_shape=[
            jax.ShapeDtypeStruct((Bc, _N + _S, _H), f32),
            jax.ShapeDtypeStruct((Bc, _L, _H), f32),
            jax.ShapeDtypeStruct((Bc, _N + _S, 1), f32),
            jax.ShapeDtypeStruct((Bc, _L, 1), f32),
            jax.ShapeDtypeStruct((Bc, _H), f32),
            jax.ShapeDtypeStruct((Bc, _TOK, _TOK), f32),
        ],
    )(nb2, st2, ln2, sl2, hs2, rt2, xc, yc, xr, yr,
      W_agent, b_agent.reshape(1, _H), type_emb, W_static,
      b_static.reshape(1, _H), W_lane, b_lane.reshape(1, _H), W_sl,
      b_sl.reshape(1, _H), unknown_sl.reshape(1, _H), traffic_emb, W_route,
      b_route.reshape(1, _H), W_pos, b_pos.reshape(1, _H))

    enc_a, ln_enc, mask_a_f, ln_valid_f, rcond, tdist = out
    mask_a = mask_a_f[:, :, 0] > 0.5
    ln_valid = ln_valid_f[:, :, 0] > 0.5
    return (enc_a, ln_enc, mask_a, ln_valid, rcond, tdist)
